# Initial kernel scaffold; baseline (speedup 1.0000x reference)
import jax
import jax.numpy as jnp
from jax.experimental import pallas as pl


def kernel(x, edge_index, edge_attr, node_ids, batch, params):
    raise NotImplementedError("write your pallas kernel here")


# speedup vs baseline: 2.8084x; 2.8084x over previous
import functools
import numpy as np
import jax
import jax.numpy as jnp
from jax import lax
from jax.experimental import pallas as pl
from jax.experimental.pallas import tpu as pltpu
from jax.experimental.pallas import tpu_sc as plsc

NN = 10000
NPAD = 10240
EE = 320000
HD = 128
NROI = 268
NRPAD = 272
NCOMM = 7
NB = 16
EPS = 1e-5
BNS = float(1.0 / np.sqrt(1.0 + EPS))
SQN = float(np.sqrt(float(NN)))

NTILE = 16
CH = 128
CPW = 160
EPW = CH * CPW
EPAD = EPW * NTILE
HW = 64
R2D = NPAD // 128

F32 = jnp.float32
I32 = jnp.int32

_dot = functools.partial(jnp.dot, preferred_element_type=F32)
_dotx = functools.partial(jnp.dot, preferred_element_type=F32,
                          precision=lax.Precision.HIGHEST)


def _sc_edge_body(ht2, src3, dst3, ea3, zs, out, src_v, dst_v, ea_v,
                  buf, zbuf, acc, sem):
    cid = lax.axis_index("c")
    sid = lax.axis_index("s")
    htc = ht2.at[cid]
    pltpu.sync_copy(src3.at[sid], src_v)
    pltpu.sync_copy(dst3.at[sid], dst_v)
    pltpu.sync_copy(ea3.at[sid], ea_v)
    pltpu.sync_copy(zs, zbuf)
    for j in range(5):
        pltpu.sync_copy(zbuf, acc.at[pl.ds(sid * 640 + j * 128, 128)])
    plsc.subcore_barrier()

    def chunk_body(i, carry):
        pltpu.async_copy(htc.at[src_v.at[i]], buf, sem).wait()

        def edge_body(j, c2):
            ii = jnp.full((16,), i, I32)
            jj = jnp.full((16,), j, I32)
            eab = plsc.load_gather(ea_v, [ii, jj])
            for v in range(HW // 16):
                sl = pl.ds(v * 16, 16)
                buf[j, sl] = buf[j, sl] * eab
            return c2

        lax.fori_loop(0, CH, edge_body, 0, unroll=False)
        pltpu.sync_copy(buf, acc.at[dst_v.at[i]], add=True)
        return carry

    lax.fori_loop(0, CPW, chunk_body, 0, unroll=False)
    plsc.subcore_barrier()
    for j in range(5):
        sl = pl.ds(sid * 640 + j * 128, 128)
        pltpu.sync_copy(acc.at[sl], zbuf)
        pltpu.sync_copy(zbuf, out.at[cid, sl])


def _sc_edge_pass(ht2, src3, dst3, ea3, zs):
    mesh = plsc.VectorSubcoreMesh(core_axis_name="c", subcore_axis_name="s",
                                  num_cores=2, num_subcores=16)
    fn = pl.kernel(
        _sc_edge_body,
        out_type=jax.ShapeDtypeStruct((2, NPAD, HW), F32),
        mesh=mesh,
        scratch_types=[
            pltpu.VMEM((CPW, CH), I32),
            pltpu.VMEM((CPW, CH), I32),
            pltpu.VMEM((CPW, CH), F32),
            pltpu.VMEM((CH, HW), F32),
            pltpu.VMEM((128, HW), F32),
            pltpu.VMEM_SHARED((NPAD, HW), F32),
            pltpu.SemaphoreType.DMA,
        ],
        compiler_params=pltpu.CompilerParams(needs_layout_passes=False,
                                             use_tc_tiling_on_sc=False),
    )
    return fn(ht2, src3, dst3, ea3, zs)


def _tc_prologue_body(x_ref, win_ref, bin_ref, gin_ref, bein_ref, nids_ref,
                      h0_ref, cnt_ref):
    x = x_ref[...]
    h = _dot(x, win_ref[...]) + bin_ref[...]
    h = h * BNS * gin_ref[...] + bein_ref[...]
    h0_ref[...] = jnp.maximum(h, 0.0)

    nids = nids_ref[...]
    iota_r = lax.broadcasted_iota(I32, (NRPAD, 1), 0)

    def cbody(r, acc):
        c = jnp.sum(jnp.where(nids == r, 1.0, 0.0))
        return acc + jnp.where(iota_r == r, c, 0.0)

    cnt_ref[...] = lax.fori_loop(0, NROI, cbody, jnp.zeros((NRPAD, 1), F32))


def _tc_prologue(xp, win, bin_, gin, bein, nids2d):
    return pl.pallas_call(
        _tc_prologue_body,
        out_shape=[
            jax.ShapeDtypeStruct((NPAD, HD), F32),
            jax.ShapeDtypeStruct((NRPAD, 1), F32),
        ],
    )(xp, win, bin_, gin, bein, nids2d)


def _tc_matmul_body(a_ref, b_ref, o_ref):
    o_ref[...] = _dot(a_ref[...], b_ref[...])


def _tc_matmul(a, b):
    return pl.pallas_call(
        _tc_matmul_body,
        out_shape=jax.ShapeDtypeStruct((a.shape[0], b.shape[1]), F32),
    )(a, b)


def _pairnorm_bn_relu(p_ref, g_ref, b_ref):
    out = jnp.concatenate([p_ref[0], p_ref[1]], axis=1)
    mean = jnp.sum(out, axis=0, keepdims=True) * (1.0 / NN)
    c = out - mean
    nrm = jnp.sqrt(jnp.sum(c * c, axis=1, keepdims=True)) + 1e-6
    o = c / nrm * SQN
    o = o * BNS * g_ref[...] + b_ref[...]
    return jnp.maximum(o, 0.0)


def _tc_mid_body(p_ref, g_ref, b_ref, kn_ref, h_ref, ht_ref):
    h = _pairnorm_bn_relu(p_ref, g_ref, b_ref)
    h_ref[...] = h
    ht_ref[...] = _dot(h, kn_ref[...])


def _tc_mid(parts, g, b, kn):
    return pl.pallas_call(
        _tc_mid_body,
        out_shape=[
            jax.ShapeDtypeStruct((NPAD, HD), F32),
            jax.ShapeDtypeStruct((NPAD, HD), F32),
        ],
    )(parts, g, b, kn)


def _tc_last_body(p_ref, g_ref, b_ref, ws1_ref, bs1_ref, ws2_ref,
                  h_ref, sc_ref):
    h = _pairnorm_bn_relu(p_ref, g_ref, b_ref)
    h_ref[...] = h
    s1 = jnp.maximum(_dot(h, ws1_ref[...]) + bs1_ref[...], 0.0)
    sc = _dot(s1, ws2_ref[...])
    sc_ref[...] = sc[:, 0:1]


def _tc_last(parts, g, b, ws1p, bs1p, ws2p):
    return pl.pallas_call(
        _tc_last_body,
        out_shape=[
            jax.ShapeDtypeStruct((NPAD, HD), F32),
            jax.ShapeDtypeStruct((NPAD, 1), F32),
        ],
    )(parts, g, b, ws1p, bs1p, ws2p)


def _tc_pool_body(h_ref, sc2_ref, bat_ref, bs2_ref, wh1_ref, bh1_ref,
                  wh2_ref, bh2_ref, out_ref, sel_ref):
    scores = sc2_ref[...] + bs2_ref[0, 0]
    bat = bat_ref[...]
    vmask = bat < NB

    bi = lax.bitcast_convert_type(scores, I32)
    bu = lax.bitcast_convert_type(scores, jnp.uint32)
    ukey = jnp.where(bi >= 0, bu | jnp.uint32(0x80000000), ~bu)
    ukey = jnp.where(vmask, ukey, jnp.uint32(0))

    masks = [bat == g for g in range(NB)]
    cnts = [jnp.sum(m.astype(I32)) for m in masks]
    kgs = [jnp.maximum(1, cnts[g] // 2) for g in range(NB)]

    p = jnp.zeros((R2D, 128), jnp.uint32)
    for bit in range(31, -1, -1):
        cand = p | jnp.uint32(1 << bit)
        ge = ukey >= cand
        accept = jnp.zeros((R2D, 128), jnp.bool_)
        for g in range(NB):
            cg = jnp.sum((masks[g] & ge).astype(I32))
            accept = accept | (masks[g] & (cg >= kgs[g]))
        p = jnp.where(accept, cand, p)
    tkey = p

    gt = vmask & (ukey > tkey)
    tie = vmask & (ukey == tkey)
    tf = tie.astype(F32)

    i128 = lax.broadcasted_iota(I32, (128, 128), 0)
    j128 = lax.broadcasted_iota(I32, (128, 128), 1)
    su = jnp.where(i128 < j128, 1.0, 0.0)
    prow = _dotx(tf, su)
    rsum = jnp.sum(tf, axis=1, keepdims=True)
    i80 = lax.broadcasted_iota(I32, (R2D, R2D), 0)
    j80 = lax.broadcasted_iota(I32, (R2D, R2D), 1)
    sl80 = jnp.where(j80 < i80, 1.0, 0.0)
    roff = _dotx(sl80, rsum)
    trank = prow + roff

    rnode = jnp.zeros((R2D, 128), F32)
    onode = jnp.zeros((R2D, 128), F32)
    off = jnp.float32(0.0)
    for g in range(NB):
        mg = masks[g]
        mgt = jnp.sum((mg & gt).astype(F32))
        rg = kgs[g].astype(F32) - mgt
        rnode = rnode + jnp.where(mg, rg, 0.0)
        onode = onode + jnp.where(mg, off, 0.0)
        off = off + jnp.sum(jnp.where(mg, tf, 0.0))
    trank_seg = trank - onode

    include = gt | (tie & (trank_seg < rnode))
    sel_ref[...] = include.astype(F32)

    def pbody(i, acc):
        brow = bat_ref[pl.ds(i, 1), :]
        srow = sel_ref[pl.ds(i, 1), :]
        hblk = h_ref[pl.ds(i * 128, 128), :]
        selg = jnp.concatenate(
            [jnp.where(brow == g, srow, 0.0) for g in range(NB)], axis=0)
        return acc + _dotx(selg, hblk)

    pooled = lax.fori_loop(0, R2D, pbody, jnp.zeros((NB, HD), F32))

    hh = jnp.maximum(_dot(pooled, wh1_ref[...]) + bh1_ref[...], 0.0)
    res = _dot(hh, wh2_ref[...])
    out_ref[...] = res[:, 0:1] + bh2_ref[0, 0]


def _tc_pool(h, sc2d, bat2d, bs2, wh1p, bh1p, wh2p, bh2):
    return pl.pallas_call(
        _tc_pool_body,
        out_shape=jax.ShapeDtypeStruct((NB, 1), F32),
        scratch_shapes=[pltpu.VMEM((R2D, 128), F32)],
    )(h, sc2d, bat2d, bs2, wh1p, bh1p, wh2p, bh2)


def kernel(x, edge_index, edge_attr, node_ids, batch, params):
    xp = jnp.zeros((NPAD, HD), F32).at[:NN].set(x.astype(F32))

    src = edge_index[0].astype(I32)
    dst = edge_index[1].astype(I32)
    ea = edge_attr.reshape(-1).astype(F32)
    pad = EPAD - EE
    src3 = jnp.concatenate([src, jnp.zeros((pad,), I32)]).reshape(NTILE, CPW, CH)
    dst3 = jnp.concatenate([dst, jnp.zeros((pad,), I32)]).reshape(NTILE, CPW, CH)
    ea3 = jnp.concatenate([ea, jnp.zeros((pad,), F32)]).reshape(NTILE, CPW, CH)
    zs = jnp.zeros((128, HW), F32)

    nids2d = jnp.full((NPAD,), NROI + 10, I32).at[:NN].set(
        node_ids.astype(I32)).reshape(R2D, 128)
    bat2d = jnp.full((NPAD,), NB, I32).at[:NN].set(
        batch.astype(I32)).reshape(R2D, 128)

    p = params
    row = lambda v: v.reshape(1, -1).astype(F32)

    h, cnts = _tc_prologue(
        xp, p['W_in'].astype(F32), row(p['b_in']), row(p['g_in']),
        row(p['be_in']), nids2d)
    counts = cnts[:NROI, 0]
    ksums = []
    for l in range(3):
        cw = jax.nn.softmax(p['roi_comm'][l], axis=-1)
        rk = jnp.einsum('rc,cio->rio', cw, p['basis'][l])
        ksums.append(jnp.einsum('r,rio->io', counts, rk))
    ht = _tc_matmul(h, ksums[0])

    ws1p = jnp.pad(p['Ws1'].astype(F32), ((0, 0), (0, HD - 64)))
    bs1p = jnp.pad(row(p['bs1']), ((0, 0), (0, HD - 64)))
    ws2p = jnp.pad(p['Ws2'].astype(F32), ((0, HD - 64), (0, 127)))
    wh1p = jnp.pad(p['Wh1'].astype(F32), ((0, 0), (0, HD - 64)))
    bh1p = jnp.pad(row(p['bh1']), ((0, 0), (0, HD - 64)))
    wh2p = jnp.pad(p['Wh2'].astype(F32), ((0, HD - 64), (0, 127)))

    kn = [ksums[1], ksums[2]]
    for l in range(3):
        ht2 = jnp.stack([ht[:, :HW], ht[:, HW:]])
        parts = _sc_edge_pass(ht2, src3, dst3, ea3, zs)
        if l < 2:
            h, ht = _tc_mid(parts, row(p['bn_g'][l]), row(p['bn_b'][l]), kn[l])
        else:
            h, sc = _tc_last(parts, row(p['bn_g'][l]), row(p['bn_b'][l]),
                             ws1p, bs1p, ws2p)

    sc2d = sc.reshape(R2D, 128)
    out = _tc_pool(h, sc2d, bat2d, row(p['bs2']), wh1p, bh1p, wh2p,
                   row(p['bh2']))
    return out


# speedup vs baseline: 2.8205x; 1.0043x over previous
import functools
import numpy as np
import jax
import jax.numpy as jnp
from jax import lax
from jax.experimental import pallas as pl
from jax.experimental.pallas import tpu as pltpu
from jax.experimental.pallas import tpu_sc as plsc

NN = 10000
NPAD = 10240
EE = 320000
HD = 128
NROI = 268
NRPAD = 272
NCOMM = 7
NB = 16
EPS = 1e-5
BNS = float(1.0 / np.sqrt(1.0 + EPS))
SQN = float(np.sqrt(float(NN)))

NTILE = 16
CH = 128
CPW = 160
EPW = CH * CPW
EPAD = EPW * NTILE
HW = 64
R2D = NPAD // 128

F32 = jnp.float32
I32 = jnp.int32

_dot = functools.partial(jnp.dot, preferred_element_type=F32)
_dotx = functools.partial(jnp.dot, preferred_element_type=F32,
                          precision=lax.Precision.HIGHEST)


def _sc_edge_body(ht2, src3, dst3, ea3, zs, out, src_v, dst_v, ea_v,
                  buf, zbuf, acc, sem):
    cid = lax.axis_index("c")
    sid = lax.axis_index("s")
    htc = ht2.at[cid]
    pltpu.sync_copy(src3.at[sid], src_v)
    pltpu.sync_copy(dst3.at[sid], dst_v)
    pltpu.sync_copy(ea3.at[sid], ea_v)
    pltpu.sync_copy(zs, zbuf)
    for j in range(5):
        pltpu.sync_copy(zbuf, acc.at[pl.ds(sid * 640 + j * 128, 128)])
    plsc.subcore_barrier()

    def chunk_body(i, carry):
        pltpu.async_copy(htc.at[src_v.at[i]], buf, sem).wait()

        def edge_body(j, c2):
            ii = jnp.full((16,), i, I32)
            jj = jnp.full((16,), j, I32)
            eab = plsc.load_gather(ea_v, [ii, jj])
            for v in range(HW // 16):
                sl = pl.ds(v * 16, 16)
                buf[j, sl] = buf[j, sl] * eab
            return c2

        lax.fori_loop(0, CH, edge_body, 0, unroll=8)
        pltpu.sync_copy(buf, acc.at[dst_v.at[i]], add=True)
        return carry

    lax.fori_loop(0, CPW, chunk_body, 0, unroll=False)
    plsc.subcore_barrier()
    for j in range(5):
        sl = pl.ds(sid * 640 + j * 128, 128)
        pltpu.sync_copy(acc.at[sl], zbuf)
        pltpu.sync_copy(zbuf, out.at[cid, sl])


def _sc_edge_pass(ht2, src3, dst3, ea3, zs):
    mesh = plsc.VectorSubcoreMesh(core_axis_name="c", subcore_axis_name="s",
                                  num_cores=2, num_subcores=16)
    fn = pl.kernel(
        _sc_edge_body,
        out_type=jax.ShapeDtypeStruct((2, NPAD, HW), F32),
        mesh=mesh,
        scratch_types=[
            pltpu.VMEM((CPW, CH), I32),
            pltpu.VMEM((CPW, CH), I32),
            pltpu.VMEM((CPW, CH), F32),
            pltpu.VMEM((CH, HW), F32),
            pltpu.VMEM((128, HW), F32),
            pltpu.VMEM_SHARED((NPAD, HW), F32),
            pltpu.SemaphoreType.DMA,
        ],
        compiler_params=pltpu.CompilerParams(needs_layout_passes=False,
                                             use_tc_tiling_on_sc=False),
    )
    return fn(ht2, src3, dst3, ea3, zs)


def _tc_prologue_body(x_ref, win_ref, bin_ref, gin_ref, bein_ref, nids_ref,
                      h0_ref, cnt_ref):
    x = x_ref[...]
    h = _dot(x, win_ref[...]) + bin_ref[...]
    h = h * BNS * gin_ref[...] + bein_ref[...]
    h0_ref[...] = jnp.maximum(h, 0.0)

    nids = nids_ref[...]
    iota_r = lax.broadcasted_iota(I32, (NRPAD, 1), 0)

    def cbody(r, acc):
        c = jnp.sum(jnp.where(nids == r, 1.0, 0.0))
        return acc + jnp.where(iota_r == r, c, 0.0)

    cnt_ref[...] = lax.fori_loop(0, NROI, cbody, jnp.zeros((NRPAD, 1), F32))


def _tc_prologue(xp, win, bin_, gin, bein, nids2d):
    return pl.pallas_call(
        _tc_prologue_body,
        out_shape=[
            jax.ShapeDtypeStruct((NPAD, HD), F32),
            jax.ShapeDtypeStruct((NRPAD, 1), F32),
        ],
    )(xp, win, bin_, gin, bein, nids2d)


def _tc_matmul_body(a_ref, b_ref, o_ref):
    o_ref[...] = _dot(a_ref[...], b_ref[...])


def _tc_matmul(a, b):
    return pl.pallas_call(
        _tc_matmul_body,
        out_shape=jax.ShapeDtypeStruct((a.shape[0], b.shape[1]), F32),
    )(a, b)


def _pairnorm_bn_relu(p_ref, g_ref, b_ref):
    out = jnp.concatenate([p_ref[0], p_ref[1]], axis=1)
    mean = jnp.sum(out, axis=0, keepdims=True) * (1.0 / NN)
    c = out - mean
    nrm = jnp.sqrt(jnp.sum(c * c, axis=1, keepdims=True)) + 1e-6
    o = c / nrm * SQN
    o = o * BNS * g_ref[...] + b_ref[...]
    return jnp.maximum(o, 0.0)


def _tc_mid_body(p_ref, g_ref, b_ref, kn_ref, h_ref, ht_ref):
    h = _pairnorm_bn_relu(p_ref, g_ref, b_ref)
    h_ref[...] = h
    ht_ref[...] = _dot(h, kn_ref[...])


def _tc_mid(parts, g, b, kn):
    return pl.pallas_call(
        _tc_mid_body,
        out_shape=[
            jax.ShapeDtypeStruct((NPAD, HD), F32),
            jax.ShapeDtypeStruct((NPAD, HD), F32),
        ],
    )(parts, g, b, kn)


def _tc_last_body(p_ref, g_ref, b_ref, ws1_ref, bs1_ref, ws2_ref,
                  h_ref, sc_ref):
    h = _pairnorm_bn_relu(p_ref, g_ref, b_ref)
    h_ref[...] = h
    s1 = jnp.maximum(_dot(h, ws1_ref[...]) + bs1_ref[...], 0.0)
    sc = _dot(s1, ws2_ref[...])
    sc_ref[...] = sc[:, 0:1]


def _tc_last(parts, g, b, ws1p, bs1p, ws2p):
    return pl.pallas_call(
        _tc_last_body,
        out_shape=[
            jax.ShapeDtypeStruct((NPAD, HD), F32),
            jax.ShapeDtypeStruct((NPAD, 1), F32),
        ],
    )(parts, g, b, ws1p, bs1p, ws2p)


def _tc_pool_body(h_ref, sc2_ref, bat_ref, bs2_ref, wh1_ref, bh1_ref,
                  wh2_ref, bh2_ref, out_ref, sel_ref):
    scores = sc2_ref[...] + bs2_ref[0, 0]
    bat = bat_ref[...]
    vmask = bat < NB

    bi = lax.bitcast_convert_type(scores, I32)
    bu = lax.bitcast_convert_type(scores, jnp.uint32)
    ukey = jnp.where(bi >= 0, bu | jnp.uint32(0x80000000), ~bu)
    ukey = jnp.where(vmask, ukey, jnp.uint32(0))

    masks = [bat == g for g in range(NB)]
    cnts = [jnp.sum(m.astype(I32)) for m in masks]
    kgs = [jnp.maximum(1, cnts[g] // 2) for g in range(NB)]

    p = jnp.zeros((R2D, 128), jnp.uint32)
    for bit in range(31, -1, -1):
        cand = p | jnp.uint32(1 << bit)
        ge = ukey >= cand
        accept = jnp.zeros((R2D, 128), jnp.bool_)
        for g in range(NB):
            cg = jnp.sum((masks[g] & ge).astype(I32))
            accept = accept | (masks[g] & (cg >= kgs[g]))
        p = jnp.where(accept, cand, p)
    tkey = p

    gt = vmask & (ukey > tkey)
    tie = vmask & (ukey == tkey)
    tf = tie.astype(F32)

    i128 = lax.broadcasted_iota(I32, (128, 128), 0)
    j128 = lax.broadcasted_iota(I32, (128, 128), 1)
    su = jnp.where(i128 < j128, 1.0, 0.0)
    prow = _dotx(tf, su)
    rsum = jnp.sum(tf, axis=1, keepdims=True)
    i80 = lax.broadcasted_iota(I32, (R2D, R2D), 0)
    j80 = lax.broadcasted_iota(I32, (R2D, R2D), 1)
    sl80 = jnp.where(j80 < i80, 1.0, 0.0)
    roff = _dotx(sl80, rsum)
    trank = prow + roff

    rnode = jnp.zeros((R2D, 128), F32)
    onode = jnp.zeros((R2D, 128), F32)
    off = jnp.float32(0.0)
    for g in range(NB):
        mg = masks[g]
        mgt = jnp.sum((mg & gt).astype(F32))
        rg = kgs[g].astype(F32) - mgt
        rnode = rnode + jnp.where(mg, rg, 0.0)
        onode = onode + jnp.where(mg, off, 0.0)
        off = off + jnp.sum(jnp.where(mg, tf, 0.0))
    trank_seg = trank - onode

    include = gt | (tie & (trank_seg < rnode))
    sel_ref[...] = include.astype(F32)

    def pbody(i, acc):
        brow = bat_ref[pl.ds(i, 1), :]
        srow = sel_ref[pl.ds(i, 1), :]
        hblk = h_ref[pl.ds(i * 128, 128), :]
        selg = jnp.concatenate(
            [jnp.where(brow == g, srow, 0.0) for g in range(NB)], axis=0)
        return acc + _dotx(selg, hblk)

    pooled = lax.fori_loop(0, R2D, pbody, jnp.zeros((NB, HD), F32))

    hh = jnp.maximum(_dot(pooled, wh1_ref[...]) + bh1_ref[...], 0.0)
    res = _dot(hh, wh2_ref[...])
    out_ref[...] = res[:, 0:1] + bh2_ref[0, 0]


def _tc_pool(h, sc2d, bat2d, bs2, wh1p, bh1p, wh2p, bh2):
    return pl.pallas_call(
        _tc_pool_body,
        out_shape=jax.ShapeDtypeStruct((NB, 1), F32),
        scratch_shapes=[pltpu.VMEM((R2D, 128), F32)],
    )(h, sc2d, bat2d, bs2, wh1p, bh1p, wh2p, bh2)


def kernel(x, edge_index, edge_attr, node_ids, batch, params):
    xp = jnp.zeros((NPAD, HD), F32).at[:NN].set(x.astype(F32))

    src = edge_index[0].astype(I32)
    dst = edge_index[1].astype(I32)
    ea = edge_attr.reshape(-1).astype(F32)
    pad = EPAD - EE
    src3 = jnp.concatenate([src, jnp.zeros((pad,), I32)]).reshape(NTILE, CPW, CH)
    dst3 = jnp.concatenate([dst, jnp.zeros((pad,), I32)]).reshape(NTILE, CPW, CH)
    ea3 = jnp.concatenate([ea, jnp.zeros((pad,), F32)]).reshape(NTILE, CPW, CH)
    zs = jnp.zeros((128, HW), F32)

    nids2d = jnp.full((NPAD,), NROI + 10, I32).at[:NN].set(
        node_ids.astype(I32)).reshape(R2D, 128)
    bat2d = jnp.full((NPAD,), NB, I32).at[:NN].set(
        batch.astype(I32)).reshape(R2D, 128)

    p = params
    row = lambda v: v.reshape(1, -1).astype(F32)

    h, cnts = _tc_prologue(
        xp, p['W_in'].astype(F32), row(p['b_in']), row(p['g_in']),
        row(p['be_in']), nids2d)
    counts = cnts[:NROI, 0]
    ksums = []
    for l in range(3):
        cw = jax.nn.softmax(p['roi_comm'][l], axis=-1)
        rk = jnp.einsum('rc,cio->rio', cw, p['basis'][l])
        ksums.append(jnp.einsum('r,rio->io', counts, rk))
    ht = _tc_matmul(h, ksums[0])

    ws1p = jnp.pad(p['Ws1'].astype(F32), ((0, 0), (0, HD - 64)))
    bs1p = jnp.pad(row(p['bs1']), ((0, 0), (0, HD - 64)))
    ws2p = jnp.pad(p['Ws2'].astype(F32), ((0, HD - 64), (0, 127)))
    wh1p = jnp.pad(p['Wh1'].astype(F32), ((0, 0), (0, HD - 64)))
    bh1p = jnp.pad(row(p['bh1']), ((0, 0), (0, HD - 64)))
    wh2p = jnp.pad(p['Wh2'].astype(F32), ((0, HD - 64), (0, 127)))

    kn = [ksums[1], ksums[2]]
    for l in range(3):
        ht2 = jnp.stack([ht[:, :HW], ht[:, HW:]])
        parts = _sc_edge_pass(ht2, src3, dst3, ea3, zs)
        if l < 2:
            h, ht = _tc_mid(parts, row(p['bn_g'][l]), row(p['bn_b'][l]), kn[l])
        else:
            h, sc = _tc_last(parts, row(p['bn_g'][l]), row(p['bn_b'][l]),
                             ws1p, bs1p, ws2p)

    sc2d = sc.reshape(R2D, 128)
    out = _tc_pool(h, sc2d, bat2d, row(p['bs2']), wh1p, bh1p, wh2p,
                   row(p['bh2']))
    return out
